# baseline (speedup 1.0000x reference)
import jax
import jax.numpy as jnp
from jax.experimental import pallas as pl
from jax.experimental.pallas import tpu as pltpu

VOCAB = 1024
E = 256
K = 8
T = 4
H = 4 * E
VB = 256
QB = 512


def _layernorm(x, eps=1e-5):
    m = jnp.mean(x, axis=-1, keepdims=True)
    v = jnp.mean((x - m) ** 2, axis=-1, keepdims=True)
    return (x - m) / jnp.sqrt(v + eps)


def _body(emb_ref, z_ref, w1_ref, b1_ref, w2_ref, b2_ref, w3_ref, b3_ref,
          w4_ref, b4_ref, tok_ref, out_ref, mem_ref):
    i = pl.program_id(0)

    @pl.when(i < T)
    def _mlp():
        x = emb_ref[...].reshape(VB * T, E)
        h = jnp.maximum(jnp.dot(x, w1_ref[...], precision=jax.lax.Precision.HIGHEST) + b1_ref[...], 0.0)
        h = jnp.maximum(jnp.dot(h, w2_ref[...], precision=jax.lax.Precision.HIGHEST) + b2_ref[...], 0.0)
        h = jnp.maximum(jnp.dot(h, w3_ref[...], precision=jax.lax.Precision.HIGHEST) + b3_ref[...], 0.0)
        mem = jnp.dot(h, w4_ref[...], precision=jax.lax.Precision.HIGHEST) + b4_ref[...]
        mem = _layernorm(mem).reshape(VB, T, E)
        for t in range(T):
            mem_ref[t, pl.ds(i * VB, VB), :] = mem[:, t, :]

    @pl.when(i >= T)
    def _attn():
        q = _layernorm(z_ref[...]) * (E ** -0.5)
        qr = q.reshape(QB // T, T, E)
        outs = []
        for t in range(T):
            qt = qr[:, t, :]
            mt = mem_ref[t]
            s = jax.lax.dot_general(qt, mt, (((1,), (1,)), ((), ())))
            p = jnp.exp(s - jnp.max(s, axis=-1, keepdims=True))
            p = p / jnp.sum(p, axis=-1, keepdims=True)
            tok_ref[0, t, :] = jnp.argmax(p, axis=-1).astype(jnp.int32)
            outs.append(jnp.dot(p, mt))
        out_ref[...] = jnp.stack(outs, axis=1).reshape(QB, E)


def kernel(z, embeddings, W1, b1, W2, b2, W3, b3, W4, b4):
    n = z.shape[0]
    emb3 = embeddings.reshape(VOCAB, T, E)
    b1r, b2r, b3r = b1.reshape(1, H), b2.reshape(1, H), b3.reshape(1, H)
    b4r = b4.reshape(1, E)
    nq = n // QB

    tok_t, z_q = pl.pallas_call(
        _body,
        grid=(T + nq,),
        in_specs=[
            pl.BlockSpec((VB, T, E), lambda i: (jnp.minimum(i, T - 1), 0, 0)),
            pl.BlockSpec((QB, E), lambda i: (jnp.maximum(i - T, 0), 0)),
            pl.BlockSpec((E, H), lambda i: (0, 0)),
            pl.BlockSpec((1, H), lambda i: (0, 0)),
            pl.BlockSpec((H, H), lambda i: (0, 0)),
            pl.BlockSpec((1, H), lambda i: (0, 0)),
            pl.BlockSpec((H, H), lambda i: (0, 0)),
            pl.BlockSpec((1, H), lambda i: (0, 0)),
            pl.BlockSpec((H, E), lambda i: (0, 0)),
            pl.BlockSpec((1, E), lambda i: (0, 0)),
        ],
        out_specs=[
            pl.BlockSpec((1, T, QB // T), lambda i: (jnp.maximum(i - T, 0), 0, 0)),
            pl.BlockSpec((QB, E), lambda i: (jnp.maximum(i - T, 0), 0)),
        ],
        out_shape=[
            jax.ShapeDtypeStruct((nq, T, QB // T), jnp.int32),
            jax.ShapeDtypeStruct((n, E), jnp.float32),
        ],
        scratch_shapes=[pltpu.VMEM((T, VOCAB, E), jnp.float32)],
    )(emb3, z, W1, b1r, W2, b2r, W3, b3r, W4, b4r)

    tokens = tok_t.transpose(0, 2, 1).reshape(n)
    return (tokens, z_q)


# speedup vs baseline: 2.3631x; 2.3631x over previous
import jax
import jax.numpy as jnp
from jax.experimental import pallas as pl
from jax.experimental.pallas import tpu as pltpu

VOCAB = 1024
E = 256
K = 8
T = 4
H = 4 * E
VB = 256
NB = VOCAB // VB


def _layernorm(x, eps=1e-5):
    m = jnp.mean(x, axis=-1, keepdims=True)
    v = jnp.mean((x - m) ** 2, axis=-1, keepdims=True)
    return (x - m) / jnp.sqrt(v + eps)


def _body(emb_ref, z_ref, w1_ref, b1_ref, w2_ref, b2_ref, w3_ref, b3_ref,
          w4_ref, b4_ref, tok_ref, out_ref,
          q_ref, acc_ref, mx_ref, l_ref, idx_ref):
    i = pl.program_id(0)
    nq = q_ref.shape[1]

    @pl.when(i == 0)
    def _init():
        zr = z_ref[...].reshape(nq, T, E)
        for t in range(T):
            q_ref[t] = _layernorm(zr[:, t, :]) * (E ** -0.5)
        acc_ref[...] = jnp.zeros_like(acc_ref)
        mx_ref[...] = jnp.full_like(mx_ref, -jnp.inf)
        l_ref[...] = jnp.zeros_like(l_ref)
        idx_ref[...] = jnp.zeros_like(idx_ref)

    x = emb_ref[...].reshape(VB * T, E)
    h = jnp.maximum(jnp.dot(x, w1_ref[...]) + b1_ref[...], 0.0)
    h = jnp.maximum(jnp.dot(h, w2_ref[...]) + b2_ref[...], 0.0)
    h = jnp.maximum(jnp.dot(h, w3_ref[...]) + b3_ref[...], 0.0)
    mem = jnp.dot(h, w4_ref[...]) + b4_ref[...]
    mem = _layernorm(mem).reshape(VB, T, E)

    for t in range(T):
        mt = mem[:, t, :]
        qt = q_ref[t]
        s = jax.lax.dot_general(qt, mt, (((1,), (1,)), ((), ())))
        smax = jnp.max(s, axis=-1, keepdims=True)
        sarg = jnp.argmax(s, axis=-1, keepdims=True).astype(jnp.int32)
        run_mx = mx_ref[t]
        new_mx = jnp.maximum(run_mx, smax)
        better = smax > run_mx
        idx_ref[t] = jnp.where(better, sarg + i * VB, idx_ref[t])
        mx_ref[t] = new_mx
        alpha = jnp.exp(run_mx - new_mx)
        p = jnp.exp(s - new_mx)
        l_ref[t] = l_ref[t] * alpha + jnp.sum(p, axis=-1, keepdims=True)
        acc_ref[t] = acc_ref[t] * alpha + jnp.dot(p, mt)

    @pl.when(i == NB - 1)
    def _finalize():
        outs = []
        for t in range(T):
            outs.append(acc_ref[t] / l_ref[t])
            tok_ref[t, :] = idx_ref[t].reshape(nq)
        out_ref[...] = jnp.stack(outs, axis=1).reshape(nq * T, E)


def kernel(z, embeddings, W1, b1, W2, b2, W3, b3, W4, b4):
    n = z.shape[0]
    nq = n // T
    emb3 = embeddings.reshape(VOCAB, T, E)
    b1r, b2r, b3r = b1.reshape(1, H), b2.reshape(1, H), b3.reshape(1, H)
    b4r = b4.reshape(1, E)

    tok_t, z_q = pl.pallas_call(
        _body,
        grid=(NB,),
        in_specs=[
            pl.BlockSpec((VB, T, E), lambda i: (i, 0, 0)),
            pl.BlockSpec((n, E), lambda i: (0, 0)),
            pl.BlockSpec((E, H), lambda i: (0, 0)),
            pl.BlockSpec((1, H), lambda i: (0, 0)),
            pl.BlockSpec((H, H), lambda i: (0, 0)),
            pl.BlockSpec((1, H), lambda i: (0, 0)),
            pl.BlockSpec((H, H), lambda i: (0, 0)),
            pl.BlockSpec((1, H), lambda i: (0, 0)),
            pl.BlockSpec((H, E), lambda i: (0, 0)),
            pl.BlockSpec((1, E), lambda i: (0, 0)),
        ],
        out_specs=[
            pl.BlockSpec((T, nq), lambda i: (0, 0)),
            pl.BlockSpec((n, E), lambda i: (0, 0)),
        ],
        out_shape=[
            jax.ShapeDtypeStruct((T, nq), jnp.int32),
            jax.ShapeDtypeStruct((n, E), jnp.float32),
        ],
        scratch_shapes=[
            pltpu.VMEM((T, nq, E), jnp.float32),
            pltpu.VMEM((T, nq, E), jnp.float32),
            pltpu.VMEM((T, nq, 1), jnp.float32),
            pltpu.VMEM((T, nq, 1), jnp.float32),
            pltpu.VMEM((T, nq, 1), jnp.int32),
        ],
    )(emb3, z, W1, b1r, W2, b2r, W3, b3r, W4, b4r)

    tokens = tok_t.T.reshape(n)
    return (tokens, z_q)


# speedup vs baseline: 3.6450x; 1.5425x over previous
import jax
import jax.numpy as jnp
from jax.experimental import pallas as pl
from jax.experimental.pallas import tpu as pltpu

VOCAB = 1024
E = 256
K = 8
T = 4
H = 4 * E
VB = 256
QB = 2048


def _layernorm(x, eps=1e-5):
    m = jnp.mean(x, axis=-1, keepdims=True)
    v = jnp.mean((x - m) ** 2, axis=-1, keepdims=True)
    return (x - m) / jnp.sqrt(v + eps)


def _body(emb_ref, z_ref, w1_ref, b1_ref, w2_ref, b2_ref, w3_ref, b3_ref,
          w4_ref, b4_ref, tok_ref, out_ref, mem_ref):
    i = pl.program_id(0)

    @pl.when(i < T)
    def _mlp():
        x = emb_ref[...].reshape(VB * T, E)
        h = jnp.maximum(jnp.dot(x, w1_ref[...]) + b1_ref[...], 0.0)
        h = jnp.maximum(jnp.dot(h, w2_ref[...]) + b2_ref[...], 0.0)
        h = jnp.maximum(jnp.dot(h, w3_ref[...]) + b3_ref[...], 0.0)
        mem = jnp.dot(h, w4_ref[...]) + b4_ref[...]
        mem = _layernorm(mem).reshape(VB, T, E)
        for t in range(T):
            mem_ref[t, pl.ds(i * VB, VB), :] = mem[:, t, :]

    @pl.when(i >= T)
    def _attn():
        q = _layernorm(z_ref[...]) * (E ** -0.5)
        qr = q.reshape(QB // T, T, E)
        outs = []
        for t in range(T):
            qt = qr[:, t, :]
            mt = mem_ref[t]
            s = jax.lax.dot_general(qt, mt, (((1,), (1,)), ((), ())))
            p = jnp.exp(s - jnp.max(s, axis=-1, keepdims=True))
            p = p / jnp.sum(p, axis=-1, keepdims=True)
            tok_ref[0, t, :] = jnp.argmax(p, axis=-1).astype(jnp.int32)
            outs.append(jnp.dot(p, mt))
        out_ref[...] = jnp.stack(outs, axis=1).reshape(QB, E)


def kernel(z, embeddings, W1, b1, W2, b2, W3, b3, W4, b4):
    n = z.shape[0]
    emb3 = embeddings.reshape(VOCAB, T, E)
    b1r, b2r, b3r = b1.reshape(1, H), b2.reshape(1, H), b3.reshape(1, H)
    b4r = b4.reshape(1, E)
    nq = n // QB

    tok_t, z_q = pl.pallas_call(
        _body,
        grid=(T + nq,),
        in_specs=[
            pl.BlockSpec((VB, T, E), lambda i: (jnp.minimum(i, T - 1), 0, 0)),
            pl.BlockSpec((QB, E), lambda i: (jnp.maximum(i - T, 0), 0)),
            pl.BlockSpec((E, H), lambda i: (0, 0)),
            pl.BlockSpec((1, H), lambda i: (0, 0)),
            pl.BlockSpec((H, H), lambda i: (0, 0)),
            pl.BlockSpec((1, H), lambda i: (0, 0)),
            pl.BlockSpec((H, H), lambda i: (0, 0)),
            pl.BlockSpec((1, H), lambda i: (0, 0)),
            pl.BlockSpec((H, E), lambda i: (0, 0)),
            pl.BlockSpec((1, E), lambda i: (0, 0)),
        ],
        out_specs=[
            pl.BlockSpec((1, T, QB // T), lambda i: (jnp.maximum(i - T, 0), 0, 0)),
            pl.BlockSpec((QB, E), lambda i: (jnp.maximum(i - T, 0), 0)),
        ],
        out_shape=[
            jax.ShapeDtypeStruct((nq, T, QB // T), jnp.int32),
            jax.ShapeDtypeStruct((n, E), jnp.float32),
        ],
        scratch_shapes=[pltpu.VMEM((T, VOCAB, E), jnp.float32)],
    )(emb3, z, W1, b1r, W2, b2r, W3, b3r, W4, b4r)

    tokens = tok_t.transpose(0, 2, 1).reshape(n)
    return (tokens, z_q)
